# baseline (speedup 1.0000x reference)
import functools

import jax
import jax.numpy as jnp
from jax.experimental import pallas as pl
from jax.experimental.pallas import tpu as pltpu

DIM = 2048
ND = 8
RANK = 64
NTOK = 16384
BLK = 1024
NDR = ND * RANK
GRID = NTOK // BLK


def _body(ids_ref, x_ref, u_ref, v_ref, s_ref, out_ref, reg_ref,
          vcat_ref, ucat_ref, cnt_ref, regd_ref):
    i = pl.program_id(0)
    dom_row = jax.lax.broadcasted_iota(jnp.int32, (1, ND), 1)

    @pl.when(i == 0)
    def _init():
        row = jax.lax.broadcasted_iota(jnp.int32, (RANK, RANK), 0)
        col = jax.lax.broadcasted_iota(jnp.int32, (RANK, RANK), 1)
        eye = (row == col).astype(jnp.float32)
        dn = (((0,), (0,)), ((), ()))
        regd = jnp.zeros((1, ND), dtype=jnp.float32)
        for d in range(ND):
            ud = u_ref[d]
            vd = v_ref[d]
            sl = pl.ds(d * RANK, RANK)
            ucat_ref[:, sl] = ud.astype(jnp.bfloat16)
            vcat_ref[:, sl] = vd.astype(jnp.bfloat16)
            gu = jax.lax.dot_general(ud, ud, dn,
                                     preferred_element_type=jnp.float32)
            gv = jax.lax.dot_general(vd, vd, dn,
                                     preferred_element_type=jnp.float32)
            reg_d = (jnp.mean((gu - eye) ** 2) + jnp.mean((gv - eye) ** 2)
                     + 0.1 * jnp.mean(jnp.abs(s_ref[0, sl])))
            regd = regd + jnp.where(dom_row == d, reg_d, 0.0)
        regd_ref[...] = regd
        cnt_ref[...] = jnp.zeros((1, ND), dtype=jnp.float32)

    x = x_ref[...]
    xb = x.astype(jnp.bfloat16)
    z = jnp.dot(xb, vcat_ref[...], preferred_element_type=jnp.float32)
    z = z * s_ref[...]
    dom = ids_ref[...]
    col_dom = jax.lax.broadcasted_iota(jnp.int32, (1, NDR), 1) // RANK
    z = jnp.where(dom == col_dom, z, 0.0).astype(jnp.bfloat16)
    proj = jax.lax.dot_general(z, ucat_ref[...], (((1,), (1,)), ((), ())),
                               preferred_element_type=jnp.float32)
    out_ref[...] = x + proj

    cnt_ref[...] += jnp.sum((dom == dom_row).astype(jnp.float32), axis=0,
                            keepdims=True)

    @pl.when(i == GRID - 1)
    def _fin():
        present = (cnt_ref[...] > 0).astype(jnp.float32)
        reg = jnp.sum(present * regd_ref[...]) / ND
        reg_ref[...] = jnp.reshape(reg, (1, 1))


@jax.jit
def kernel(feats, domain_ids, U, V, s):
    s_flat = s.reshape(1, NDR)
    ids2 = domain_ids.reshape(NTOK, 1)

    out, reg = pl.pallas_call(
        _body,
        grid=(GRID,),
        in_specs=[
            pl.BlockSpec((BLK, 1), lambda i: (i, 0)),
            pl.BlockSpec((BLK, DIM), lambda i: (i, 0)),
            pl.BlockSpec((ND, DIM, RANK), lambda i: (0, 0, 0)),
            pl.BlockSpec((ND, DIM, RANK), lambda i: (0, 0, 0)),
            pl.BlockSpec((1, NDR), lambda i: (0, 0)),
        ],
        out_specs=[
            pl.BlockSpec((BLK, DIM), lambda i: (i, 0)),
            pl.BlockSpec((1, 1), lambda i: (0, 0)),
        ],
        out_shape=[
            jax.ShapeDtypeStruct((NTOK, DIM), jnp.float32),
            jax.ShapeDtypeStruct((1, 1), jnp.float32),
        ],
        scratch_shapes=[
            pltpu.VMEM((DIM, NDR), jnp.bfloat16),
            pltpu.VMEM((DIM, NDR), jnp.bfloat16),
            pltpu.VMEM((1, ND), jnp.float32),
            pltpu.VMEM((1, ND), jnp.float32),
        ],
        compiler_params=pltpu.CompilerParams(
            dimension_semantics=("arbitrary",),
        ),
    )(ids2, feats, U, V, s_flat)

    return out, reg.reshape(1)


# speedup vs baseline: 1.0984x; 1.0984x over previous
import functools

import jax
import jax.numpy as jnp
from jax.experimental import pallas as pl
from jax.experimental.pallas import tpu as pltpu

DIM = 2048
ND = 8
RANK = 64
NTOK = 16384
BLK = 1024
NDR = ND * RANK
GRID = NTOK // BLK


def _body(ids_ref, x_ref, vcat_ref, ustack_ref, s_ref, out_ref, reg_ref,
          cnt_ref, regd_ref):
    i = pl.program_id(0)

    x = x_ref[...]
    xb = x.astype(jnp.bfloat16)
    z = jnp.dot(xb, vcat_ref[...], preferred_element_type=jnp.float32)
    z = z * s_ref[...]
    dom = ids_ref[...]
    col_dom = jax.lax.broadcasted_iota(jnp.int32, (1, NDR), 1) // RANK
    z = jnp.where(dom == col_dom, z, 0.0).astype(jnp.bfloat16)
    proj = jnp.dot(z, ustack_ref[...], preferred_element_type=jnp.float32)
    out_ref[...] = x + proj

    dom_row = jax.lax.broadcasted_iota(jnp.int32, (1, ND), 1)
    blk_cnt = jnp.sum((dom == dom_row).astype(jnp.float32), axis=0,
                      keepdims=True)

    @pl.when(i == 0)
    def _init():
        cnt_ref[...] = blk_cnt
        row = jax.lax.broadcasted_iota(jnp.int32, (RANK, RANK), 0)
        col = jax.lax.broadcasted_iota(jnp.int32, (RANK, RANK), 1)
        eye = (row == col).astype(jnp.float32)
        regd = jnp.zeros((1, ND), dtype=jnp.float32)
        for d in range(ND):
            vd = vcat_ref[:, d * RANK:(d + 1) * RANK]
            ud = ustack_ref[d * RANK:(d + 1) * RANK, :]
            gv = jax.lax.dot_general(vd, vd, (((0,), (0,)), ((), ())),
                                     preferred_element_type=jnp.float32)
            gu = jax.lax.dot_general(ud, ud, (((1,), (1,)), ((), ())),
                                     preferred_element_type=jnp.float32)
            reg_d = (jnp.mean((gu - eye) ** 2) + jnp.mean((gv - eye) ** 2)
                     + 0.1 * jnp.mean(jnp.abs(s_ref[0, d * RANK:(d + 1) * RANK])))
            regd = regd + jnp.where(dom_row == d, reg_d, 0.0)
        regd_ref[...] = regd

    @pl.when(i > 0)
    def _acc():
        cnt_ref[...] += blk_cnt

    @pl.when(i == GRID - 1)
    def _fin():
        present = (cnt_ref[...] > 0).astype(jnp.float32)
        reg = jnp.sum(present * regd_ref[...]) / ND
        reg_ref[...] = jnp.reshape(reg, (1, 1))


@jax.jit
def kernel(feats, domain_ids, U, V, s):
    vcat = jnp.transpose(V, (1, 0, 2)).reshape(DIM, NDR).astype(jnp.bfloat16)
    ustack = jnp.transpose(U, (0, 2, 1)).reshape(NDR, DIM).astype(jnp.bfloat16)
    s_flat = s.reshape(1, NDR)
    ids2 = domain_ids.reshape(NTOK, 1)

    out, reg = pl.pallas_call(
        _body,
        grid=(GRID,),
        in_specs=[
            pl.BlockSpec((BLK, 1), lambda i: (i, 0)),
            pl.BlockSpec((BLK, DIM), lambda i: (i, 0)),
            pl.BlockSpec((DIM, NDR), lambda i: (0, 0)),
            pl.BlockSpec((NDR, DIM), lambda i: (0, 0)),
            pl.BlockSpec((1, NDR), lambda i: (0, 0)),
        ],
        out_specs=[
            pl.BlockSpec((BLK, DIM), lambda i: (i, 0)),
            pl.BlockSpec((1, 1), lambda i: (0, 0)),
        ],
        out_shape=[
            jax.ShapeDtypeStruct((NTOK, DIM), jnp.float32),
            jax.ShapeDtypeStruct((1, 1), jnp.float32),
        ],
        scratch_shapes=[
            pltpu.VMEM((1, ND), jnp.float32),
            pltpu.VMEM((1, ND), jnp.float32),
        ],
        compiler_params=pltpu.CompilerParams(
            dimension_semantics=("arbitrary",),
            allow_input_fusion=[False, False, True, True, False],
        ),
    )(ids2, feats, vcat, ustack, s_flat)

    return out, reg.reshape(1)


# speedup vs baseline: 1.1854x; 1.0792x over previous
import functools

import jax
import jax.numpy as jnp
from jax.experimental import pallas as pl
from jax.experimental.pallas import tpu as pltpu

DIM = 2048
ND = 8
RANK = 64
NTOK = 16384
BLK = 1024
NDR = ND * RANK
GRID = NTOK // BLK
F8 = jnp.float8_e4m3fn


def _body(ids_ref, x_ref, vcat_ref, ustack_ref, s_ref, out_ref, reg_ref,
          vcat8_ref, ustack8_ref, cnt_ref, regd_ref):
    i = pl.program_id(0)
    dom = ids_ref[...]
    dom_row = jax.lax.broadcasted_iota(jnp.int32, (1, ND), 1)
    blk_cnt = jnp.sum((dom == dom_row).astype(jnp.float32), axis=0,
                      keepdims=True)

    @pl.when(i == 0)
    def _init():
        cnt_ref[...] = blk_cnt
        vcat8_ref[...] = (vcat_ref[...] * 16.0).astype(F8)
        ustack8_ref[...] = ustack_ref[...].astype(F8)
        row = jax.lax.broadcasted_iota(jnp.int32, (RANK, RANK), 0)
        col = jax.lax.broadcasted_iota(jnp.int32, (RANK, RANK), 1)
        eye = (row == col).astype(jnp.float32)
        regd = jnp.zeros((1, ND), dtype=jnp.float32)
        for d in range(ND):
            vd = vcat_ref[:, d * RANK:(d + 1) * RANK]
            ud = ustack_ref[d * RANK:(d + 1) * RANK, :]
            gv = jax.lax.dot_general(vd, vd, (((0,), (0,)), ((), ())),
                                     preferred_element_type=jnp.float32)
            gu = jax.lax.dot_general(ud, ud, (((1,), (1,)), ((), ())),
                                     preferred_element_type=jnp.float32)
            reg_d = (jnp.mean((gu - eye) ** 2) + jnp.mean((gv - eye) ** 2)
                     + 1.6 * jnp.mean(jnp.abs(s_ref[0, d * RANK:(d + 1) * RANK])))
            regd = regd + jnp.where(dom_row == d, reg_d, 0.0)
        regd_ref[...] = regd

    @pl.when(i > 0)
    def _acc():
        cnt_ref[...] += blk_cnt

    x = x_ref[...]
    xb = x.astype(F8)
    z = jnp.dot(xb, vcat8_ref[...], preferred_element_type=jnp.float32)
    z = z * s_ref[...]
    col_dom = jax.lax.broadcasted_iota(jnp.int32, (1, NDR), 1) // RANK
    z = jnp.where(dom == col_dom, z, 0.0).astype(F8)
    proj = jnp.dot(z, ustack8_ref[...], preferred_element_type=jnp.float32)
    out_ref[...] = x + proj

    @pl.when(i == GRID - 1)
    def _fin():
        present = (cnt_ref[...] > 0).astype(jnp.float32)
        reg = jnp.sum(present * regd_ref[...]) / ND
        reg_ref[...] = jnp.reshape(reg, (1, 1))


@jax.jit
def kernel(feats, domain_ids, U, V, s):
    vcat = jnp.transpose(V, (1, 0, 2)).reshape(DIM, NDR).astype(jnp.bfloat16)
    ustack = jnp.transpose(U, (0, 2, 1)).reshape(NDR, DIM).astype(jnp.bfloat16)
    s_flat = (s / 16.0).reshape(1, NDR)
    ids2 = domain_ids.reshape(NTOK, 1)

    out, reg = pl.pallas_call(
        _body,
        grid=(GRID,),
        in_specs=[
            pl.BlockSpec((BLK, 1), lambda i: (i, 0)),
            pl.BlockSpec((BLK, DIM), lambda i: (i, 0)),
            pl.BlockSpec((DIM, NDR), lambda i: (0, 0)),
            pl.BlockSpec((NDR, DIM), lambda i: (0, 0)),
            pl.BlockSpec((1, NDR), lambda i: (0, 0)),
        ],
        out_specs=[
            pl.BlockSpec((BLK, DIM), lambda i: (i, 0)),
            pl.BlockSpec((1, 1), lambda i: (0, 0)),
        ],
        out_shape=[
            jax.ShapeDtypeStruct((NTOK, DIM), jnp.float32),
            jax.ShapeDtypeStruct((1, 1), jnp.float32),
        ],
        scratch_shapes=[
            pltpu.VMEM((DIM, NDR), F8),
            pltpu.VMEM((NDR, DIM), F8),
            pltpu.VMEM((1, ND), jnp.float32),
            pltpu.VMEM((1, ND), jnp.float32),
        ],
        compiler_params=pltpu.CompilerParams(
            dimension_semantics=("arbitrary",),
        ),
    )(ids2, feats, vcat, ustack, s_flat)

    return out, reg.reshape(1)


# speedup vs baseline: 1.2251x; 1.0336x over previous
import functools

import jax
import jax.numpy as jnp
from jax.experimental import pallas as pl
from jax.experimental.pallas import tpu as pltpu

DIM = 2048
ND = 8
RANK = 64
NTOK = 16384
BLK = 1024
NDR = ND * RANK
GRID = NTOK // BLK
F8 = jnp.float8_e4m3fn


def _body(ids_ref, x_ref, vcat_ref, ustack_ref, s_ref, out_ref, reg_ref,
          vcat8_ref, ustack8_ref, cnt_ref, gv_ref, gu_ref):
    i = pl.program_id(0)
    dom = ids_ref[...]
    dom_row = jax.lax.broadcasted_iota(jnp.int32, (1, ND), 1)
    blk_cnt = jnp.sum((dom == dom_row).astype(jnp.float32), axis=0,
                      keepdims=True)

    @pl.when(i == 0)
    def _init():
        cnt_ref[...] = blk_cnt
        vcat8_ref[...] = (vcat_ref[...] * 16.0).astype(F8)
        ustack8_ref[...] = ustack_ref[...].astype(F8)

    @pl.when(i > 0)
    def _acc():
        cnt_ref[...] += blk_cnt

    @pl.when(i == GRID - 2)
    def _grams():
        vc = vcat_ref[...]
        us = ustack_ref[...]
        gv_ref[...] = jax.lax.dot_general(vc, vc, (((0,), (0,)), ((), ())),
                                          preferred_element_type=jnp.float32)
        gu_ref[...] = jax.lax.dot_general(us, us, (((1,), (1,)), ((), ())),
                                          preferred_element_type=jnp.float32)

    x = x_ref[...]
    xb = x.astype(F8)
    z = jnp.dot(xb, vcat8_ref[...], preferred_element_type=jnp.float32)
    z = z * s_ref[...]
    col_dom = jax.lax.broadcasted_iota(jnp.int32, (1, NDR), 1) // RANK
    z = jnp.where(dom == col_dom, z, 0.0).astype(F8)
    proj = jnp.dot(z, ustack8_ref[...], preferred_element_type=jnp.float32)
    out_ref[...] = x + proj

    @pl.when(i == GRID - 1)
    def _fin():
        row = jax.lax.broadcasted_iota(jnp.int32, (RANK, RANK), 0)
        col = jax.lax.broadcasted_iota(jnp.int32, (RANK, RANK), 1)
        eye = (row == col).astype(jnp.float32)
        regd = jnp.zeros((1, ND), dtype=jnp.float32)
        for d in range(ND):
            sl = slice(d * RANK, (d + 1) * RANK)
            reg_d = (jnp.mean((gu_ref[sl, sl] - eye) ** 2)
                     + jnp.mean((gv_ref[sl, sl] - eye) ** 2)
                     + 1.6 * jnp.mean(jnp.abs(s_ref[0, sl])))
            regd = regd + jnp.where(dom_row == d, reg_d, 0.0)
        present = (cnt_ref[...] > 0).astype(jnp.float32)
        reg = jnp.sum(present * regd) / ND
        reg_ref[...] = jnp.reshape(reg, (1, 1))


@jax.jit
def kernel(feats, domain_ids, U, V, s):
    vcat = jnp.transpose(V, (1, 0, 2)).reshape(DIM, NDR).astype(jnp.bfloat16)
    ustack = jnp.transpose(U, (0, 2, 1)).reshape(NDR, DIM).astype(jnp.bfloat16)
    s_flat = (s / 16.0).reshape(1, NDR)
    ids2 = domain_ids.reshape(NTOK, 1)

    out, reg = pl.pallas_call(
        _body,
        grid=(GRID,),
        in_specs=[
            pl.BlockSpec((BLK, 1), lambda i: (i, 0)),
            pl.BlockSpec((BLK, DIM), lambda i: (i, 0)),
            pl.BlockSpec((DIM, NDR), lambda i: (0, 0)),
            pl.BlockSpec((NDR, DIM), lambda i: (0, 0)),
            pl.BlockSpec((1, NDR), lambda i: (0, 0)),
        ],
        out_specs=[
            pl.BlockSpec((BLK, DIM), lambda i: (i, 0)),
            pl.BlockSpec((1, 1), lambda i: (0, 0)),
        ],
        out_shape=[
            jax.ShapeDtypeStruct((NTOK, DIM), jnp.float32),
            jax.ShapeDtypeStruct((1, 1), jnp.float32),
        ],
        scratch_shapes=[
            pltpu.VMEM((DIM, NDR), F8),
            pltpu.VMEM((NDR, DIM), F8),
            pltpu.VMEM((1, ND), jnp.float32),
            pltpu.VMEM((NDR, NDR), jnp.float32),
            pltpu.VMEM((NDR, NDR), jnp.float32),
        ],
        compiler_params=pltpu.CompilerParams(
            dimension_semantics=("arbitrary",),
        ),
    )(ids2, feats, vcat, ustack, s_flat)

    return out, reg.reshape(1)
